# Initial kernel scaffold; baseline (speedup 1.0000x reference)
import jax
import jax.numpy as jnp
from jax.experimental import pallas as pl


def kernel(x, token_table, pos_table):
    raise NotImplementedError("write your pallas kernel here")


# speedup vs baseline: 3.3997x; 3.3997x over previous
import functools

import jax
import jax.numpy as jnp
from jax import lax
from jax.experimental import pallas as pl
from jax.experimental.pallas import tpu as pltpu
from jax.experimental.pallas import tpu_sc as plsc

LANES = 16


@functools.partial(jax.jit, static_argnames=("B", "T", "D"))
def _embed(x_flat, token_table, pos_table, B, T, D):
    NC, NS = 2, 16
    NW = NC * NS
    W = 128
    B_PER_W = B // NW
    NTC = T // W

    mesh = plsc.VectorSubcoreMesh(core_axis_name="c", subcore_axis_name="s")

    @functools.partial(
        pl.kernel,
        mesh=mesh,
        compiler_params=pltpu.CompilerParams(use_tc_tiling_on_sc=False),
        out_type=jax.ShapeDtypeStruct((B * T, D), jnp.float32),
        scratch_types=[
            pltpu.VMEM((W,), jnp.int32),
            pltpu.VMEM((W, D), jnp.float32),
            pltpu.VMEM((W, D), jnp.float32),
            pltpu.VMEM((W, D), jnp.float32),
            pltpu.SemaphoreType.DMA,
        ],
    )
    def k(x_hbm, tok_hbm, pos_hbm, out_hbm, idx_v, rows_v, pos_v, out_v, sem):
        wid = lax.axis_index("s") * NC + lax.axis_index("c")
        b0 = wid * B_PER_W

        @pl.loop(0, NTC)
        def _(tc):
            pltpu.sync_copy(pos_hbm.at[pl.ds(tc * W, W)], pos_v)

            @pl.loop(0, B_PER_W)
            def _(i):
                row0 = (b0 + i) * T + tc * W
                pltpu.sync_copy(x_hbm.at[pl.ds(row0, W)], idx_v)
                pltpu.async_copy(tok_hbm.at[idx_v], rows_v, sem).wait()

                @pl.loop(0, W)
                def _(r):
                    for c in range(0, D, LANES):
                        out_v[r, pl.ds(c, LANES)] = (
                            rows_v[r, pl.ds(c, LANES)]
                            + pos_v[r, pl.ds(c, LANES)]
                        )

                pltpu.sync_copy(out_v, out_hbm.at[pl.ds(row0, W)])

    return k(x_flat, token_table, pos_table)


def kernel(x, token_table, pos_table):
    B, T = x.shape
    D = token_table.shape[1]
    out = _embed(x.reshape(-1), token_table, pos_table, B, T, D)
    return out.reshape(B, T, D)


# speedup vs baseline: 3.4019x; 1.0006x over previous
import functools

import jax
import jax.numpy as jnp
from jax import lax
from jax.experimental import pallas as pl
from jax.experimental.pallas import tpu as pltpu
from jax.experimental.pallas import tpu_sc as plsc

LANES = 16


@functools.partial(jax.jit, static_argnames=("B", "T", "D"))
def _embed(x_flat, token_table, pos_table, B, T, D):
    NC, NS = 2, 16
    NW = NC * NS
    W = 128
    B_PER_W = B // NW
    NTC = T // W

    mesh = plsc.VectorSubcoreMesh(core_axis_name="c", subcore_axis_name="s")

    PACK = 128 // D
    WO = W // PACK

    @functools.partial(
        pl.kernel,
        mesh=mesh,
        compiler_params=pltpu.CompilerParams(use_tc_tiling_on_sc=False),
        out_type=jax.ShapeDtypeStruct((B * T // PACK, PACK * D), jnp.float32),
        scratch_types=[
            pltpu.VMEM((W,), jnp.int32),
            pltpu.VMEM((W, D), jnp.float32),
            pltpu.VMEM((W, D), jnp.float32),
            pltpu.VMEM((WO, PACK * D), jnp.float32),
            pltpu.SemaphoreType.DMA,
        ],
    )
    def k(x_hbm, tok_hbm, pos_hbm, out_hbm, idx_v, rows_v, pos_v, out_v, sem):
        wid = lax.axis_index("s") * NC + lax.axis_index("c")
        b0 = wid * B_PER_W

        @pl.loop(0, NTC)
        def _(tc):
            pltpu.sync_copy(pos_hbm.at[pl.ds(tc * W, W)], pos_v)

            @pl.loop(0, B_PER_W)
            def _(i):
                row0 = (b0 + i) * T + tc * W
                pltpu.sync_copy(x_hbm.at[pl.ds(row0, W)], idx_v)
                pltpu.async_copy(tok_hbm.at[idx_v], rows_v, sem).wait()

                @pl.loop(0, WO)
                def _(j):
                    for g in range(0, PACK * D, LANES):
                        r = PACK * j + g // D
                        c = g % D
                        out_v[j, pl.ds(g, LANES)] = (
                            rows_v[r, pl.ds(c, LANES)]
                            + pos_v[r, pl.ds(c, LANES)]
                        )

                pltpu.sync_copy(
                    out_v, out_hbm.at[pl.ds(row0 // PACK, WO)]
                )

    return k(x_flat, token_table, pos_table)


def kernel(x, token_table, pos_table):
    B, T = x.shape
    D = token_table.shape[1]
    out = _embed(x.reshape(-1), token_table, pos_table, B, T, D)
    return out.reshape(B, T, D)
